# baseline (speedup 1.0000x reference)
import functools

import jax
import jax.numpy as jnp
from jax import lax
from jax.experimental import pallas as pl
from jax.experimental.pallas import tpu as pltpu
import jax.experimental.pallas.tpu_sc as plsc

N = 4096
K = 16
DG = 256
DPHI = 20
HID = 150
NC = 2
NS = 16
NW = NC * NS
B = N * K
TW = 384
CH = 128
MB = 256
PB = MB * K
DPAD = 64
NSLAB = 1


def _sc_gather(table, idx):
    nb = idx.shape[0]
    b_per_w = nb // NW
    n_chunks = b_per_w // CH
    mesh = plsc.VectorSubcoreMesh(core_axis_name="c", subcore_axis_name="s")

    @functools.partial(
        pl.kernel,
        out_type=jax.ShapeDtypeStruct((nb, TW), jnp.float32),
        mesh=mesh,
        scratch_types=[
            pltpu.VMEM((CH,), jnp.int32),
            pltpu.VMEM((CH,), jnp.int32),
            pltpu.VMEM((CH, TW), jnp.float32),
            pltpu.VMEM((CH, TW), jnp.float32),
            pltpu.SemaphoreType.DMA,
            pltpu.SemaphoreType.DMA,
            pltpu.SemaphoreType.DMA,
            pltpu.SemaphoreType.DMA,
        ],
    )
    def gather_kernel(table_hbm, idx_hbm, out_hbm, idx0, idx1, rows0, rows1,
                      gsem0, gsem1, ssem0, ssem1):
        wid = lax.axis_index("s") * NC + lax.axis_index("c")
        idx_v = (idx0, idx1)
        rows_v = (rows0, rows1)
        gsem = (gsem0, gsem1)
        ssem = (ssem0, ssem1)
        gathers = [None, None]
        stores = [None, None]
        for c in range(n_chunks + 1):
            if c < n_chunks:
                k = c % 2
                if stores[k] is not None:
                    stores[k].wait()
                    stores[k] = None
                base = wid * b_per_w + c * CH
                pltpu.sync_copy(idx_hbm.at[pl.ds(base, CH)], idx_v[k])
                gathers[k] = pltpu.async_copy(
                    table_hbm.at[idx_v[k]], rows_v[k], gsem[k])
            if c >= 1:
                k = (c - 1) % 2
                base = wid * b_per_w + (c - 1) * CH
                gathers[k].wait()
                stores[k] = pltpu.async_copy(
                    rows_v[k], out_hbm.at[pl.ds(base, CH)], ssem[k])
        for s in stores:
            if s is not None:
                s.wait()

    return gather_kernel(table, idx)


def _tc_main_body(g_ref, s_ref, gen_ref, u_ref, dist_ref, spk_ref,
                  w1a_ref, w1b_ref, w1c_ref, w1d_ref, emb_ref, embg_ref,
                  w2_ref, w3_ref, b1_ref, b2_ref, b3_ref,
                  tlo_ref, thi_ref, e_ref, et_ref, ohk_ref, out_ref):
    f32 = jnp.float32
    dot = functools.partial(jnp.dot, preferred_element_type=f32)

    g = g_ref[...]
    gj = u_ref[:, :DG]
    sj = u_ref[:, DG:DG + 1]
    E = e_ref[...]

    gexp = dot(E, g)
    h1 = dot(gj, w1b_ref[...]) + dot(gexp * gj, w1c_ref[...])

    dist = dist_ref[...]
    ohd = jnp.logical_and(dist >= tlo_ref[...], dist < thi_ref[...])
    iota16 = lax.broadcasted_iota(jnp.int32, (PB, K), 1)
    ohs = iota16 == spk_ref[...] + 9
    oh2 = ohd.astype(f32) + ohs.astype(f32)

    ohg = (lax.broadcasted_iota(jnp.int32, (MB, 8), 1)
           == gen_ref[...]).astype(f32)
    a32 = (dot(g, w1a_ref[...]) + b1_ref[...]
           + dot(ohg, dot(embg_ref[...], w1d_ref[...])))
    tab16 = dot(emb_ref[...], w1d_ref[...])

    h1 = jnp.maximum(h1 + dot(E, a32) + dot(oh2, tab16), 0.0)
    h2 = jnp.maximum(dot(h1, w2_ref[...]) + b2_ref[...], 0.0)
    s_ij = jnp.sum(h2 * w3_ref[...], axis=1, keepdims=True)
    s_i = dot(E, s_ref[...])
    score = s_ij + b3_ref[...] + s_i + sj

    smat = dot(et_ref[...], score * ohk_ref[...])
    m = jnp.maximum(jnp.max(smat, axis=1, keepdims=True), 0.0)
    e_exp = jnp.exp(smat - m)
    eps = jnp.exp(-m)
    den = jnp.sum(e_exp, axis=1, keepdims=True) + eps
    out_ref[...] = jnp.concatenate([e_exp, eps], axis=1) / den


def _tc_main(g_i, sc_m, genres2, u, dist2, spk2,
             w1a, w1b, w1c, w1d, emb, embg, w2, w3r, b1r, b2r, b3r,
             tlo, thi, e_mat, et_mat, ohk, interpret=False):
    nm = g_i.shape[0]
    grid = nm // MB
    full = lambda i: (0, 0)
    return pl.pallas_call(
        _tc_main_body,
        grid=(grid,),
        in_specs=[
            pl.BlockSpec((MB, DG), lambda i: (i, 0)),
            pl.BlockSpec((MB, 1), lambda i: (i, 0)),
            pl.BlockSpec((MB, 1), lambda i: (i, 0)),
            pl.BlockSpec((PB, TW), lambda i: (i, 0)),
            pl.BlockSpec((PB, 1), lambda i: (i, 0)),
            pl.BlockSpec((PB, 1), lambda i: (i, 0)),
            pl.BlockSpec((DG, HID), full),
            pl.BlockSpec((DG, HID), full),
            pl.BlockSpec((DG, HID), full),
            pl.BlockSpec((DPAD, HID), full),
            pl.BlockSpec((K, DPAD), full),
            pl.BlockSpec((8, DPAD), full),
            pl.BlockSpec((HID, HID), full),
            pl.BlockSpec((1, HID), full),
            pl.BlockSpec((1, HID), full),
            pl.BlockSpec((1, HID), full),
            pl.BlockSpec((1, 1), full),
            pl.BlockSpec((1, K), full),
            pl.BlockSpec((1, K), full),
            pl.BlockSpec((PB, MB), full),
            pl.BlockSpec((MB, PB), full),
            pl.BlockSpec((PB, K), full),
        ],
        out_specs=pl.BlockSpec((MB, K + 1), lambda i: (i, 0)),
        out_shape=jax.ShapeDtypeStruct((nm, K + 1), jnp.float32),
        compiler_params=pltpu.CompilerParams(
            dimension_semantics=("arbitrary",)),
        interpret=interpret,
    )(g_i, sc_m, genres2, u, dist2, spk2,
      w1a, w1b, w1c, w1d, emb, embg, w2, w3r, b1r, b2r, b3r,
      tlo, thi, e_mat, et_mat, ohk)


def kernel(g_i, mention_scores, antecedent_ids, distances, genres, speakers,
           dist_emb, genre_emb, speaker_emb, W1, b1, W2, b2, W3, b3):
    f32 = jnp.float32
    g_i = g_i.astype(f32)
    sc_m = mention_scores.astype(f32)

    ant_flat = antecedent_ids.reshape(-1).astype(jnp.int32)
    dist2 = distances.reshape(-1, 1).astype(jnp.int32)
    spk2 = speakers.reshape(-1, 1).astype(jnp.int32)
    genres2 = genres.reshape(-1, 1).astype(jnp.int32)

    table = jnp.concatenate(
        [g_i, sc_m, jnp.zeros((N, TW - DG - 1), f32)], axis=1)

    w1a = W1[:DG].astype(f32)
    w1b = W1[DG:2 * DG].astype(f32)
    w1c = W1[2 * DG:3 * DG].astype(f32)
    w1d = jnp.concatenate(
        [W1[3 * DG:].astype(f32), jnp.zeros((4, HID), f32)], axis=0)
    emb = jnp.zeros((K, DPAD), f32)
    emb = emb.at[:9, :DPHI].set(dist_emb.astype(f32))
    emb = emb.at[9:12, 2 * DPHI:3 * DPHI].set(speaker_emb.astype(f32))
    embg = jnp.zeros((8, DPAD), f32)
    embg = embg.at[:, DPHI:2 * DPHI].set(genre_emb.astype(f32))
    big = jnp.int32(2**30)
    tlo = jnp.asarray([[0, 1, 2, 3, 4, 8, 16, 32, 64,
                        big, big, big, big, big, big, big]], jnp.int32)
    thi = jnp.asarray([[1, 2, 3, 4, 8, 16, 32, 64, big,
                        0, 0, 0, 0, 0, 0, 0]], jnp.int32)

    e_mat = jnp.repeat(jnp.eye(MB, dtype=f32), K, axis=0)
    et_mat = e_mat.T
    ohk = jnp.tile(jnp.eye(K, dtype=f32), (MB, 1))

    b1r = b1.reshape(1, HID).astype(f32)
    b2r = b2.reshape(1, HID).astype(f32)
    b3r = b3.reshape(1, 1).astype(f32)
    w3r = W3.reshape(1, HID).astype(f32)

    w2f = W2.astype(f32)
    nms = N // NSLAB
    nps = B // NSLAB
    outs = []
    for s in range(NSLAB):
        u = _sc_gather(table, lax.dynamic_slice_in_dim(ant_flat, s * nps, nps))
        outs.append(_tc_main(
            g_i[s * nms:(s + 1) * nms], sc_m[s * nms:(s + 1) * nms],
            genres2[s * nms:(s + 1) * nms], u,
            dist2[s * nps:(s + 1) * nps], spk2[s * nps:(s + 1) * nps],
            w1a, w1b, w1c, w1d, emb, embg, w2f, w3r, b1r, b2r, b3r,
            tlo, thi, e_mat, et_mat, ohk))
    return jnp.concatenate(outs, axis=0)


# speedup vs baseline: 1.0791x; 1.0791x over previous
import functools

import jax
import jax.numpy as jnp
from jax import lax
from jax.experimental import pallas as pl
from jax.experimental.pallas import tpu as pltpu
import jax.experimental.pallas.tpu_sc as plsc

N = 4096
K = 16
DG = 256
DPHI = 20
HID = 150
NC = 2
NS = 16
NW = NC * NS
B = N * K
TW = 256
CH = 128
MB = 128
PB = MB * K
DPAD = 64
NSLAB = 1


def _sc_gather(table, idx):
    nb = idx.shape[0]
    b_per_w = nb // NW
    n_chunks = b_per_w // CH
    mesh = plsc.VectorSubcoreMesh(core_axis_name="c", subcore_axis_name="s")

    @functools.partial(
        pl.kernel,
        out_type=jax.ShapeDtypeStruct((nb, TW), jnp.float32),
        mesh=mesh,
        scratch_types=[
            pltpu.VMEM((CH,), jnp.int32),
            pltpu.VMEM((CH, TW), jnp.float32),
            pltpu.SemaphoreType.DMA,
        ],
    )
    def gather_kernel(table_hbm, idx_hbm, out_hbm, idx_v, rows_v, sem):
        wid = lax.axis_index("s") * NC + lax.axis_index("c")

        def body(c, carry):
            base = wid * b_per_w + c * CH
            pltpu.sync_copy(idx_hbm.at[pl.ds(base, CH)], idx_v)
            pltpu.async_copy(table_hbm.at[idx_v], rows_v, sem).wait()
            pltpu.sync_copy(rows_v, out_hbm.at[pl.ds(base, CH)])
            return carry

        lax.fori_loop(0, n_chunks, body, None)

    return gather_kernel(table, idx)


def _tc_main_body(g_ref, s_ref, gen_ref, u_ref, dist_ref, spk_ref,
                  w1a_ref, w1b_ref, w1c_ref, w1d_ref, emb_ref, embg_ref,
                  w2_ref, w3_ref, b1_ref, b2_ref, b3_ref,
                  tlo_ref, thi_ref, e_ref, et_ref, ohk_ref, out_ref):
    f32 = jnp.float32
    dot = functools.partial(jnp.dot, preferred_element_type=f32)

    g = g_ref[...]
    w = lax.bitcast_convert_type(u_ref[:, :DG // 2], jnp.int32)
    ghi = lax.bitcast_convert_type(
        jnp.bitwise_and(w, jnp.int32(-65536)), f32)
    glo = lax.bitcast_convert_type(jnp.left_shift(w, 16), f32)
    gj = jnp.concatenate([ghi, glo], axis=1)
    gjb = gj.astype(jnp.bfloat16)
    sj = u_ref[:, DG // 2:DG // 2 + 1]
    E = e_ref[...]

    gexp = dot(E, g)
    prod = (gexp * gj).astype(jnp.bfloat16)
    h1 = dot(gjb, w1b_ref[...]) + dot(prod, w1c_ref[...])

    dist = dist_ref[...]
    ohd = jnp.logical_and(dist >= tlo_ref[...], dist < thi_ref[...])
    iota16 = lax.broadcasted_iota(jnp.int32, (PB, K), 1)
    ohs = iota16 == spk_ref[...] + 9
    oh2 = ohd.astype(f32) + ohs.astype(f32)

    ohg = (lax.broadcasted_iota(jnp.int32, (MB, 8), 1)
           == gen_ref[...]).astype(f32)
    a32 = (dot(g, w1a_ref[...]) + b1_ref[...]
           + dot(ohg, dot(embg_ref[...], w1d_ref[...])))
    tab16 = dot(emb_ref[...], w1d_ref[...])

    h1 = jnp.maximum(h1 + dot(E, a32) + dot(oh2, tab16), 0.0)
    h2 = jnp.maximum(dot(h1.astype(jnp.bfloat16), w2_ref[...])
                     + b2_ref[...], 0.0)
    s_ij = jnp.sum(h2 * w3_ref[...], axis=1, keepdims=True)
    s_i = dot(E, s_ref[...])
    score = s_ij + b3_ref[...] + s_i + sj

    smat = dot(et_ref[...], score * ohk_ref[...])
    m = jnp.maximum(jnp.max(smat, axis=1, keepdims=True), 0.0)
    e_exp = jnp.exp(smat - m)
    eps = jnp.exp(-m)
    den = jnp.sum(e_exp, axis=1, keepdims=True) + eps
    out_ref[...] = jnp.concatenate([e_exp, eps], axis=1) / den


def _tc_main(g_i, sc_m, genres2, u, dist2, spk2,
             w1a, w1b, w1c, w1d, emb, embg, w2, w3r, b1r, b2r, b3r,
             tlo, thi, e_mat, et_mat, ohk, interpret=False):
    nm = g_i.shape[0]
    grid = nm // MB
    full = lambda i: (0, 0)
    return pl.pallas_call(
        _tc_main_body,
        grid=(grid,),
        in_specs=[
            pl.BlockSpec((MB, DG), lambda i: (i, 0)),
            pl.BlockSpec((MB, 1), lambda i: (i, 0)),
            pl.BlockSpec((MB, 1), lambda i: (i, 0)),
            pl.BlockSpec((PB, TW), lambda i: (i, 0)),
            pl.BlockSpec((PB, 1), lambda i: (i, 0)),
            pl.BlockSpec((PB, 1), lambda i: (i, 0)),
            pl.BlockSpec((DG, HID), full),
            pl.BlockSpec((DG, HID), full),
            pl.BlockSpec((DG, HID), full),
            pl.BlockSpec((DPAD, HID), full),
            pl.BlockSpec((K, DPAD), full),
            pl.BlockSpec((8, DPAD), full),
            pl.BlockSpec((HID, HID), full),
            pl.BlockSpec((1, HID), full),
            pl.BlockSpec((1, HID), full),
            pl.BlockSpec((1, HID), full),
            pl.BlockSpec((1, 1), full),
            pl.BlockSpec((1, K), full),
            pl.BlockSpec((1, K), full),
            pl.BlockSpec((PB, MB), full),
            pl.BlockSpec((MB, PB), full),
            pl.BlockSpec((PB, K), full),
        ],
        out_specs=pl.BlockSpec((MB, K + 1), lambda i: (i, 0)),
        out_shape=jax.ShapeDtypeStruct((nm, K + 1), jnp.float32),
        compiler_params=pltpu.CompilerParams(
            dimension_semantics=("arbitrary",)),
        interpret=interpret,
    )(g_i, sc_m, genres2, u, dist2, spk2,
      w1a, w1b, w1c, w1d, emb, embg, w2, w3r, b1r, b2r, b3r,
      tlo, thi, e_mat, et_mat, ohk)


def kernel(g_i, mention_scores, antecedent_ids, distances, genres, speakers,
           dist_emb, genre_emb, speaker_emb, W1, b1, W2, b2, W3, b3):
    f32 = jnp.float32
    g_i = g_i.astype(f32)
    sc_m = mention_scores.astype(f32)

    ant_flat = antecedent_ids.reshape(-1).astype(jnp.int32)
    dist2 = distances.reshape(-1, 1).astype(jnp.int32)
    spk2 = speakers.reshape(-1, 1).astype(jnp.int32)
    genres2 = genres.reshape(-1, 1).astype(jnp.int32)

    gb = g_i.astype(jnp.bfloat16)
    hi = lax.bitcast_convert_type(gb[:, :DG // 2], jnp.uint16).astype(jnp.uint32)
    lo = lax.bitcast_convert_type(gb[:, DG // 2:], jnp.uint16).astype(jnp.uint32)
    packed = lax.bitcast_convert_type((hi << 16) | lo, f32)
    table = jnp.concatenate(
        [packed, sc_m, jnp.zeros((N, TW - DG // 2 - 1), f32)], axis=1)

    w1a = W1[:DG].astype(f32)
    w1b = W1[DG:2 * DG].astype(jnp.bfloat16)
    w1c = W1[2 * DG:3 * DG].astype(jnp.bfloat16)
    w1d = jnp.concatenate(
        [W1[3 * DG:].astype(f32), jnp.zeros((4, HID), f32)], axis=0)
    emb = jnp.zeros((K, DPAD), f32)
    emb = emb.at[:9, :DPHI].set(dist_emb.astype(f32))
    emb = emb.at[9:12, 2 * DPHI:3 * DPHI].set(speaker_emb.astype(f32))
    embg = jnp.zeros((8, DPAD), f32)
    embg = embg.at[:, DPHI:2 * DPHI].set(genre_emb.astype(f32))
    big = jnp.int32(2**30)
    tlo = jnp.asarray([[0, 1, 2, 3, 4, 8, 16, 32, 64,
                        big, big, big, big, big, big, big]], jnp.int32)
    thi = jnp.asarray([[1, 2, 3, 4, 8, 16, 32, 64, big,
                        0, 0, 0, 0, 0, 0, 0]], jnp.int32)

    e_mat = jnp.repeat(jnp.eye(MB, dtype=f32), K, axis=0)
    et_mat = e_mat.T
    ohk = jnp.tile(jnp.eye(K, dtype=f32), (MB, 1))

    b1r = b1.reshape(1, HID).astype(f32)
    b2r = b2.reshape(1, HID).astype(f32)
    b3r = b3.reshape(1, 1).astype(f32)
    w3r = W3.reshape(1, HID).astype(f32)

    w2f = W2.astype(jnp.bfloat16)
    nms = N // NSLAB
    nps = B // NSLAB
    outs = []
    for s in range(NSLAB):
        u = _sc_gather(table, lax.dynamic_slice_in_dim(ant_flat, s * nps, nps))
        outs.append(_tc_main(
            g_i[s * nms:(s + 1) * nms], sc_m[s * nms:(s + 1) * nms],
            genres2[s * nms:(s + 1) * nms], u,
            dist2[s * nps:(s + 1) * nps], spk2[s * nps:(s + 1) * nps],
            w1a, w1b, w1c, w1d, emb, embg, w2f, w3r, b1r, b2r, b3r,
            tlo, thi, e_mat, et_mat, ohk))
    return jnp.concatenate(outs, axis=0)


# speedup vs baseline: 1.2133x; 1.1244x over previous
import functools

import jax
import jax.numpy as jnp
from jax import lax
from jax.experimental import pallas as pl
from jax.experimental.pallas import tpu as pltpu
import jax.experimental.pallas.tpu_sc as plsc

N = 4096
K = 16
DG = 256
DPHI = 20
HID = 150
NC = 2
NS = 16
NW = NC * NS
B = N * K
TW = 256
CH = 128
MB = 256
PB = MB * K
DPAD = 64
NSLAB = 1


def _sc_gather(table, idx):
    nb = idx.shape[0]
    b_per_w = nb // NW
    n_chunks = b_per_w // CH
    mesh = plsc.VectorSubcoreMesh(core_axis_name="c", subcore_axis_name="s")

    @functools.partial(
        pl.kernel,
        out_type=jax.ShapeDtypeStruct((nb, TW), jnp.float32),
        mesh=mesh,
        scratch_types=[
            pltpu.VMEM((CH,), jnp.int32),
            pltpu.VMEM((CH, TW), jnp.float32),
            pltpu.SemaphoreType.DMA,
        ],
    )
    def gather_kernel(table_hbm, idx_hbm, out_hbm, idx_v, rows_v, sem):
        wid = lax.axis_index("s") * NC + lax.axis_index("c")

        def body(c, carry):
            base = wid * b_per_w + c * CH
            pltpu.sync_copy(idx_hbm.at[pl.ds(base, CH)], idx_v)
            pltpu.async_copy(table_hbm.at[idx_v], rows_v, sem).wait()
            pltpu.sync_copy(rows_v, out_hbm.at[pl.ds(base, CH)])
            return carry

        lax.fori_loop(0, n_chunks, body, None)

    return gather_kernel(table, idx)


def _tc_main_body(g_ref, s_ref, gen_ref, u_ref, dist_ref, spk_ref,
                  w1a_ref, w1b_ref, w1c_ref, w1d_ref, emb_ref, embg_ref,
                  w2_ref, w3_ref, b1_ref, b2_ref, b3_ref,
                  tlo_ref, thi_ref, e_ref, et_ref, ohk_ref, out_ref):
    f32 = jnp.float32
    dot = functools.partial(jnp.dot, preferred_element_type=f32)

    g = g_ref[...]
    w = lax.bitcast_convert_type(u_ref[:, :DG // 2], jnp.int32)
    ghi = lax.bitcast_convert_type(
        jnp.bitwise_and(w, jnp.int32(-65536)), f32)
    glo = lax.bitcast_convert_type(jnp.left_shift(w, 16), f32)
    gj = jnp.concatenate([ghi, glo], axis=1)
    gjb = gj.astype(jnp.bfloat16)
    sj = u_ref[:, DG // 2:DG // 2 + 1]
    E = e_ref[...]

    gexp = dot(E, g)
    prod = (gexp * gj).astype(jnp.bfloat16)
    h1 = dot(gjb, w1b_ref[...]) + dot(prod, w1c_ref[...])

    dist = dist_ref[...]
    ohd = jnp.logical_and(dist >= tlo_ref[...], dist < thi_ref[...])
    iota16 = lax.broadcasted_iota(jnp.int32, (PB, K), 1)
    ohs = iota16 == spk_ref[...] + 9
    oh2 = ohd.astype(f32) + ohs.astype(f32)

    ohg = (lax.broadcasted_iota(jnp.int32, (MB, 8), 1)
           == gen_ref[...]).astype(f32)
    a32 = (dot(g, w1a_ref[...]) + b1_ref[...]
           + dot(ohg, dot(embg_ref[...], w1d_ref[...])))
    tab16 = dot(emb_ref[...], w1d_ref[...])

    h1 = jnp.maximum(h1 + dot(E, a32) + dot(oh2, tab16), 0.0)
    h2 = jnp.maximum(dot(h1.astype(jnp.bfloat16), w2_ref[...])
                     + b2_ref[...], 0.0)
    s_ij = jnp.sum(h2 * w3_ref[...], axis=1, keepdims=True)
    s_i = dot(E, s_ref[...])
    score = s_ij + b3_ref[...] + s_i + sj

    smat = dot(et_ref[...], score * ohk_ref[...])
    m = jnp.maximum(jnp.max(smat, axis=1, keepdims=True), 0.0)
    e_exp = jnp.exp(smat - m)
    eps = jnp.exp(-m)
    den = jnp.sum(e_exp, axis=1, keepdims=True) + eps
    out_ref[...] = jnp.concatenate([e_exp, eps], axis=1) / den


def _tc_main(g_i, sc_m, genres2, u, dist2, spk2,
             w1a, w1b, w1c, w1d, emb, embg, w2, w3r, b1r, b2r, b3r,
             tlo, thi, e_mat, et_mat, ohk, interpret=False):
    nm = g_i.shape[0]
    grid = nm // MB
    full = lambda i: (0, 0)
    return pl.pallas_call(
        _tc_main_body,
        grid=(grid,),
        in_specs=[
            pl.BlockSpec((MB, DG), lambda i: (i, 0)),
            pl.BlockSpec((MB, 1), lambda i: (i, 0)),
            pl.BlockSpec((MB, 1), lambda i: (i, 0)),
            pl.BlockSpec((PB, TW), lambda i: (i, 0)),
            pl.BlockSpec((PB, 1), lambda i: (i, 0)),
            pl.BlockSpec((PB, 1), lambda i: (i, 0)),
            pl.BlockSpec((DG, HID), full),
            pl.BlockSpec((DG, HID), full),
            pl.BlockSpec((DG, HID), full),
            pl.BlockSpec((DPAD, HID), full),
            pl.BlockSpec((K, DPAD), full),
            pl.BlockSpec((8, DPAD), full),
            pl.BlockSpec((HID, HID), full),
            pl.BlockSpec((1, HID), full),
            pl.BlockSpec((1, HID), full),
            pl.BlockSpec((1, HID), full),
            pl.BlockSpec((1, 1), full),
            pl.BlockSpec((1, K), full),
            pl.BlockSpec((1, K), full),
            pl.BlockSpec((PB, MB), full),
            pl.BlockSpec((MB, PB), full),
            pl.BlockSpec((PB, K), full),
        ],
        out_specs=pl.BlockSpec((MB, K + 1), lambda i: (i, 0)),
        out_shape=jax.ShapeDtypeStruct((nm, K + 1), jnp.float32),
        compiler_params=pltpu.CompilerParams(
            dimension_semantics=("arbitrary",)),
        interpret=interpret,
    )(g_i, sc_m, genres2, u, dist2, spk2,
      w1a, w1b, w1c, w1d, emb, embg, w2, w3r, b1r, b2r, b3r,
      tlo, thi, e_mat, et_mat, ohk)


def kernel(g_i, mention_scores, antecedent_ids, distances, genres, speakers,
           dist_emb, genre_emb, speaker_emb, W1, b1, W2, b2, W3, b3):
    f32 = jnp.float32
    g_i = g_i.astype(f32)
    sc_m = mention_scores.astype(f32)

    ant_flat = antecedent_ids.reshape(-1).astype(jnp.int32)
    dist2 = distances.reshape(-1, 1).astype(jnp.int32)
    spk2 = speakers.reshape(-1, 1).astype(jnp.int32)
    genres2 = genres.reshape(-1, 1).astype(jnp.int32)

    gb = g_i.astype(jnp.bfloat16)
    hi = lax.bitcast_convert_type(gb[:, :DG // 2], jnp.uint16).astype(jnp.uint32)
    lo = lax.bitcast_convert_type(gb[:, DG // 2:], jnp.uint16).astype(jnp.uint32)
    packed = lax.bitcast_convert_type((hi << 16) | lo, f32)
    table = jnp.concatenate(
        [packed, sc_m, jnp.zeros((N, TW - DG // 2 - 1), f32)], axis=1)

    w1a = W1[:DG].astype(f32)
    w1b = W1[DG:2 * DG].astype(jnp.bfloat16)
    w1c = W1[2 * DG:3 * DG].astype(jnp.bfloat16)
    w1d = jnp.concatenate(
        [W1[3 * DG:].astype(f32), jnp.zeros((4, HID), f32)], axis=0)
    emb = jnp.zeros((K, DPAD), f32)
    emb = emb.at[:9, :DPHI].set(dist_emb.astype(f32))
    emb = emb.at[9:12, 2 * DPHI:3 * DPHI].set(speaker_emb.astype(f32))
    embg = jnp.zeros((8, DPAD), f32)
    embg = embg.at[:, DPHI:2 * DPHI].set(genre_emb.astype(f32))
    big = jnp.int32(2**30)
    tlo = jnp.asarray([[0, 1, 2, 3, 4, 8, 16, 32, 64,
                        big, big, big, big, big, big, big]], jnp.int32)
    thi = jnp.asarray([[1, 2, 3, 4, 8, 16, 32, 64, big,
                        0, 0, 0, 0, 0, 0, 0]], jnp.int32)

    e_mat = jnp.repeat(jnp.eye(MB, dtype=f32), K, axis=0)
    et_mat = e_mat.T
    ohk = jnp.tile(jnp.eye(K, dtype=f32), (MB, 1))

    b1r = b1.reshape(1, HID).astype(f32)
    b2r = b2.reshape(1, HID).astype(f32)
    b3r = b3.reshape(1, 1).astype(f32)
    w3r = W3.reshape(1, HID).astype(f32)

    w2f = W2.astype(jnp.bfloat16)
    nms = N // NSLAB
    nps = B // NSLAB
    outs = []
    for s in range(NSLAB):
        u = _sc_gather(table, lax.dynamic_slice_in_dim(ant_flat, s * nps, nps))
        outs.append(_tc_main(
            g_i[s * nms:(s + 1) * nms], sc_m[s * nms:(s + 1) * nms],
            genres2[s * nms:(s + 1) * nms], u,
            dist2[s * nps:(s + 1) * nps], spk2[s * nps:(s + 1) * nps],
            w1a, w1b, w1c, w1d, emb, embg, w2f, w3r, b1r, b2r, b3r,
            tlo, thi, e_mat, et_mat, ohk))
    return jnp.concatenate(outs, axis=0)


# speedup vs baseline: 1.2135x; 1.0002x over previous
import functools

import jax
import jax.numpy as jnp
from jax import lax
from jax.experimental import pallas as pl
from jax.experimental.pallas import tpu as pltpu
import jax.experimental.pallas.tpu_sc as plsc

N = 4096
K = 16
DG = 256
DPHI = 20
HID = 150
NC = 2
NS = 16
NW = NC * NS
B = N * K
TW = 256
CH = 128
MB = 256
PB = MB * K
DPAD = 64
NSLAB = 1


def _sc_gather(table, idx):
    nb = idx.shape[0]
    b_per_w = nb // NW
    n_chunks = b_per_w // CH
    mesh = plsc.VectorSubcoreMesh(core_axis_name="c", subcore_axis_name="s")

    @functools.partial(
        pl.kernel,
        out_type=jax.ShapeDtypeStruct((nb, TW), jnp.float32),
        mesh=mesh,
        scratch_types=[
            pltpu.VMEM((CH,), jnp.int32),
            pltpu.VMEM((CH, TW), jnp.float32),
            pltpu.SemaphoreType.DMA,
        ],
    )
    def gather_kernel(table_hbm, idx_hbm, out_hbm, idx_v, rows_v, sem):
        wid = lax.axis_index("s") * NC + lax.axis_index("c")

        def body(c, carry):
            base = wid * b_per_w + c * CH
            pltpu.sync_copy(idx_hbm.at[pl.ds(base, CH)], idx_v)
            pltpu.async_copy(table_hbm.at[idx_v], rows_v, sem).wait()
            pltpu.sync_copy(rows_v, out_hbm.at[pl.ds(base, CH)])
            return carry

        lax.fori_loop(0, n_chunks, body, None)

    return gather_kernel(table, idx)


def _tc_main_body(g_ref, s_ref, gen_ref, u_ref, dist_ref, spk_ref,
                  w1a_ref, w1b_ref, w1c_ref, w1d_ref, emb_ref, embg_ref,
                  w2_ref, w3_ref, b1_ref, b2_ref, b3_ref,
                  tlo_ref, thi_ref, e_ref, et_ref, ohk_ref, out_ref):
    f32 = jnp.float32
    dot = functools.partial(jnp.dot, preferred_element_type=f32)

    g = g_ref[...]
    w = lax.bitcast_convert_type(u_ref[:, :DG // 2], jnp.int32)
    ghi = lax.bitcast_convert_type(
        jnp.bitwise_and(w, jnp.int32(-65536)), f32)
    glo = lax.bitcast_convert_type(jnp.left_shift(w, 16), f32)
    gj = jnp.concatenate([ghi, glo], axis=1)
    gjb = gj.astype(jnp.bfloat16)
    sj = u_ref[:, DG // 2:DG // 2 + 1]
    E = e_ref[...]

    gexp = dot(E.astype(jnp.bfloat16), g.astype(jnp.bfloat16))
    prod = (gexp * gj).astype(jnp.bfloat16)
    h1 = dot(gjb, w1b_ref[...]) + dot(prod, w1c_ref[...])

    dist = dist_ref[...]
    ohd = jnp.logical_and(dist >= tlo_ref[...], dist < thi_ref[...])
    iota16 = lax.broadcasted_iota(jnp.int32, (PB, K), 1)
    ohs = iota16 == spk_ref[...] + 9
    oh2 = ohd.astype(f32) + ohs.astype(f32)

    ohg = (lax.broadcasted_iota(jnp.int32, (MB, 8), 1)
           == gen_ref[...]).astype(f32)
    a32 = (dot(g, w1a_ref[...]) + b1_ref[...]
           + dot(ohg, dot(embg_ref[...], w1d_ref[...])))
    tab16 = dot(emb_ref[...], w1d_ref[...])

    h1 = jnp.maximum(h1 + dot(E, a32) + dot(oh2, tab16), 0.0)
    h2 = jnp.maximum(dot(h1.astype(jnp.bfloat16), w2_ref[...])
                     + b2_ref[...], 0.0)
    s_ij = jnp.sum(h2 * w3_ref[...], axis=1, keepdims=True)
    s_i = dot(E, s_ref[...])
    score = s_ij + b3_ref[...] + s_i + sj

    smat = dot(et_ref[...], score * ohk_ref[...])
    m = jnp.maximum(jnp.max(smat, axis=1, keepdims=True), 0.0)
    e_exp = jnp.exp(smat - m)
    eps = jnp.exp(-m)
    den = jnp.sum(e_exp, axis=1, keepdims=True) + eps
    out_ref[...] = jnp.concatenate([e_exp, eps], axis=1) / den


def _tc_main(g_i, sc_m, genres2, u, dist2, spk2,
             w1a, w1b, w1c, w1d, emb, embg, w2, w3r, b1r, b2r, b3r,
             tlo, thi, e_mat, et_mat, ohk, interpret=False):
    nm = g_i.shape[0]
    grid = nm // MB
    full = lambda i: (0, 0)
    return pl.pallas_call(
        _tc_main_body,
        grid=(grid,),
        in_specs=[
            pl.BlockSpec((MB, DG), lambda i: (i, 0)),
            pl.BlockSpec((MB, 1), lambda i: (i, 0)),
            pl.BlockSpec((MB, 1), lambda i: (i, 0)),
            pl.BlockSpec((PB, TW), lambda i: (i, 0)),
            pl.BlockSpec((PB, 1), lambda i: (i, 0)),
            pl.BlockSpec((PB, 1), lambda i: (i, 0)),
            pl.BlockSpec((DG, HID), full),
            pl.BlockSpec((DG, HID), full),
            pl.BlockSpec((DG, HID), full),
            pl.BlockSpec((DPAD, HID), full),
            pl.BlockSpec((K, DPAD), full),
            pl.BlockSpec((8, DPAD), full),
            pl.BlockSpec((HID, HID), full),
            pl.BlockSpec((1, HID), full),
            pl.BlockSpec((1, HID), full),
            pl.BlockSpec((1, HID), full),
            pl.BlockSpec((1, 1), full),
            pl.BlockSpec((1, K), full),
            pl.BlockSpec((1, K), full),
            pl.BlockSpec((PB, MB), full),
            pl.BlockSpec((MB, PB), full),
            pl.BlockSpec((PB, K), full),
        ],
        out_specs=pl.BlockSpec((MB, K + 1), lambda i: (i, 0)),
        out_shape=jax.ShapeDtypeStruct((nm, K + 1), jnp.float32),
        compiler_params=pltpu.CompilerParams(
            dimension_semantics=("arbitrary",)),
        interpret=interpret,
    )(g_i, sc_m, genres2, u, dist2, spk2,
      w1a, w1b, w1c, w1d, emb, embg, w2, w3r, b1r, b2r, b3r,
      tlo, thi, e_mat, et_mat, ohk)


def kernel(g_i, mention_scores, antecedent_ids, distances, genres, speakers,
           dist_emb, genre_emb, speaker_emb, W1, b1, W2, b2, W3, b3):
    f32 = jnp.float32
    g_i = g_i.astype(f32)
    sc_m = mention_scores.astype(f32)

    ant_flat = antecedent_ids.reshape(-1).astype(jnp.int32)
    dist2 = distances.reshape(-1, 1).astype(jnp.int32)
    spk2 = speakers.reshape(-1, 1).astype(jnp.int32)
    genres2 = genres.reshape(-1, 1).astype(jnp.int32)

    gb = g_i.astype(jnp.bfloat16)
    hi = lax.bitcast_convert_type(gb[:, :DG // 2], jnp.uint16).astype(jnp.uint32)
    lo = lax.bitcast_convert_type(gb[:, DG // 2:], jnp.uint16).astype(jnp.uint32)
    packed = lax.bitcast_convert_type((hi << 16) | lo, f32)
    table = jnp.concatenate(
        [packed, sc_m, jnp.zeros((N, TW - DG // 2 - 1), f32)], axis=1)

    w1a = W1[:DG].astype(f32)
    w1b = W1[DG:2 * DG].astype(jnp.bfloat16)
    w1c = W1[2 * DG:3 * DG].astype(jnp.bfloat16)
    w1d = jnp.concatenate(
        [W1[3 * DG:].astype(f32), jnp.zeros((4, HID), f32)], axis=0)
    emb = jnp.zeros((K, DPAD), f32)
    emb = emb.at[:9, :DPHI].set(dist_emb.astype(f32))
    emb = emb.at[9:12, 2 * DPHI:3 * DPHI].set(speaker_emb.astype(f32))
    embg = jnp.zeros((8, DPAD), f32)
    embg = embg.at[:, DPHI:2 * DPHI].set(genre_emb.astype(f32))
    big = jnp.int32(2**30)
    tlo = jnp.asarray([[0, 1, 2, 3, 4, 8, 16, 32, 64,
                        big, big, big, big, big, big, big]], jnp.int32)
    thi = jnp.asarray([[1, 2, 3, 4, 8, 16, 32, 64, big,
                        0, 0, 0, 0, 0, 0, 0]], jnp.int32)

    e_mat = jnp.repeat(jnp.eye(MB, dtype=f32), K, axis=0)
    et_mat = e_mat.T
    ohk = jnp.tile(jnp.eye(K, dtype=f32), (MB, 1))

    b1r = b1.reshape(1, HID).astype(f32)
    b2r = b2.reshape(1, HID).astype(f32)
    b3r = b3.reshape(1, 1).astype(f32)
    w3r = W3.reshape(1, HID).astype(f32)

    w2f = W2.astype(jnp.bfloat16)
    nms = N // NSLAB
    nps = B // NSLAB
    outs = []
    for s in range(NSLAB):
        u = _sc_gather(table, lax.dynamic_slice_in_dim(ant_flat, s * nps, nps))
        outs.append(_tc_main(
            g_i[s * nms:(s + 1) * nms], sc_m[s * nms:(s + 1) * nms],
            genres2[s * nms:(s + 1) * nms], u,
            dist2[s * nps:(s + 1) * nps], spk2[s * nps:(s + 1) * nps],
            w1a, w1b, w1c, w1d, emb, embg, w2f, w3r, b1r, b2r, b3r,
            tlo, thi, e_mat, et_mat, ohk))
    return jnp.concatenate(outs, axis=0)
